# baseline (speedup 1.0000x reference)
import functools

import jax
import jax.numpy as jnp
from jax import lax
from jax.experimental import pallas as pl
from jax.experimental.pallas import tpu as pltpu
from jax.experimental.pallas import tpu_sc as plsc

B, L, V, D = 4096, 200, 1000000, 32
DP = 128
RPP = DP // D
VP = V // RPP
LANES = 16
NUM_WORKERS = 32
BPW = B // NUM_WORKERS
LPAD = 208
C0, C1 = 128, 72

RB = 256
QB = RB // RPP
NPAIR = 61
NBLK_MAIN = 2 * NPAIR * NUM_WORKERS
TAIL0 = NBLK_MAIN * RB
TAIL1 = TAIL0 + RB
TAIL2 = TAIL1 + RB
TAILR = V - TAIL2


def _repack_block(in_v, out_v, nrows):
    def body(rb, carry):
        for j in range(8):
            r = rb * 8 + j
            q = rb * 2 + (j // 4)
            o = (j % 4) * D
            out_v[q, pl.ds(o, LANES)] = in_v[r, pl.ds(0, LANES)]
            out_v[q, pl.ds(o + LANES, LANES)] = in_v[r, pl.ds(LANES, LANES)]
        return carry

    lax.fori_loop(0, nrows // 8, body, 0)


def _relayout_body(w_hbm, wlin_hbm, in_a, in_b, out_a, out_b,
                   sia, sib, soa, sob):
    c = lax.axis_index("c")
    s = lax.axis_index("s")
    wid = s * 2 + c

    def blk_of(i):
        return pl.multiple_of((wid + i * NUM_WORKERS) * RB, RB)

    def fire_in(i, in_v, sem):
        return pltpu.async_copy(w_hbm.at[pl.ds(blk_of(i), RB), :], in_v, sem)

    def wait_in(i, in_v, sem):
        pltpu.make_async_copy(w_hbm.at[pl.ds(blk_of(i), RB), :], in_v, sem).wait()

    def fire_out(i, out_v, sem):
        q0 = pl.multiple_of(blk_of(i) // RPP, QB)
        return pltpu.async_copy(out_v, wlin_hbm.at[pl.ds(q0, QB), :], sem)

    def wait_out(i, out_v, sem):
        q0 = pl.multiple_of(blk_of(i) // RPP, QB)
        pltpu.make_async_copy(
            out_v, wlin_hbm.at[pl.ds(q0, QB), :], sem).wait()

    fire_in(0, in_a, sia)
    fire_in(1, in_b, sib)

    wait_in(0, in_a, sia)
    _repack_block(in_a, out_a, RB)
    fire_in(2, in_a, sia)
    fire_out(0, out_a, soa)
    wait_in(1, in_b, sib)
    _repack_block(in_b, out_b, RB)
    fire_in(3, in_b, sib)
    fire_out(1, out_b, sob)

    def pair(p, carry):
        i0 = 2 * p
        wait_out(i0 - 2, out_a, soa)
        wait_in(i0, in_a, sia)
        _repack_block(in_a, out_a, RB)
        fire_in(i0 + 2, in_a, sia)
        fire_out(i0, out_a, soa)
        wait_out(i0 - 1, out_b, sob)
        wait_in(i0 + 1, in_b, sib)
        _repack_block(in_b, out_b, RB)
        fire_in(i0 + 3, in_b, sib)
        fire_out(i0 + 1, out_b, sob)
        return carry

    lax.fori_loop(1, NPAIR - 1, pair, 0)

    i0 = 2 * (NPAIR - 1)
    wait_out(i0 - 2, out_a, soa)
    wait_in(i0, in_a, sia)
    _repack_block(in_a, out_a, RB)
    fire_out(i0, out_a, soa)
    wait_out(i0 - 1, out_b, sob)
    wait_in(i0 + 1, in_b, sib)
    _repack_block(in_b, out_b, RB)
    fire_out(i0 + 1, out_b, sob)
    wait_out(i0, out_a, soa)
    wait_out(i0 + 1, out_b, sob)

    @pl.when(wid == 0)
    def _():
        pltpu.sync_copy(w_hbm.at[pl.ds(TAIL0, RB), :], in_a)
        _repack_block(in_a, out_a, RB)
        pltpu.sync_copy(out_a, wlin_hbm.at[pl.ds(TAIL0 // RPP, QB), :])

    @pl.when(wid == 1)
    def _():
        pltpu.sync_copy(w_hbm.at[pl.ds(TAIL1, RB), :], in_a)
        _repack_block(in_a, out_a, RB)
        pltpu.sync_copy(out_a, wlin_hbm.at[pl.ds(TAIL1 // RPP, QB), :])

    @pl.when(wid == 2)
    def _():
        pltpu.sync_copy(w_hbm.at[pl.ds(TAIL2, TAILR), :],
                        in_a.at[pl.ds(0, TAILR)])
        _repack_block(in_a, out_a, TAILR)
        pltpu.sync_copy(out_a.at[pl.ds(0, TAILR // RPP)],
                        wlin_hbm.at[pl.ds(TAIL2 // RPP, TAILR // RPP), :])


IOTA = None


def _gather_batch(wlin, qrow, rows, sem):
    cp0 = pltpu.async_copy(
        wlin.at[qrow.at[pl.ds(0, C0)]], rows.at[pl.ds(0, C0)], sem)
    cp1 = pltpu.async_copy(
        wlin.at[qrow.at[pl.ds(C0, C1)]], rows.at[pl.ds(C0, C1)], sem)
    return cp0, cp1


def _drain_batch(wlin, qrow, rows, sem):
    pltpu.make_async_copy(
        wlin.at[qrow.at[pl.ds(0, C0)]], rows.at[pl.ds(0, C0)], sem).wait()
    pltpu.make_async_copy(
        wlin.at[qrow.at[pl.ds(C0, C1)]], rows.at[pl.ds(C0, C1)], sem).wait()


def _make_qrow(idx_all, qrow, b):
    def body(t, carry):
        chunk = idx_all[b, pl.ds(t * LANES, LANES)]
        qrow[pl.ds(t * LANES, LANES)] = lax.shift_right_logical(chunk, 2)
        return carry

    lax.fori_loop(0, L // LANES, body, 0)
    chunk = idx_all[b, pl.ds(L - LANES, LANES)]
    qrow[pl.ds(L - LANES, LANES)] = lax.shift_right_logical(chunk, 2)


def _accum_batch(rows, idx_all, len_v, out_v, b):
    n = len_v[pl.ds(b, LANES)][0]
    nc = n >> 4
    rem = n & 15
    iota = lax.iota(jnp.int32, LANES)
    zero = jnp.zeros((LANES,), jnp.float32)

    def tok(i, olanes, pos, a0, a1, mf=None):
        ol = jnp.take(olanes, jnp.full((LANES,), pos, dtype=jnp.int32))
        lane0 = ol + iota
        row_i = jnp.full((LANES,), i, dtype=jnp.int32)
        v0 = plsc.load_gather(rows, [row_i, lane0])
        v1 = plsc.load_gather(rows, [row_i, lane0 + LANES])
        if mf is not None:
            v0 = v0 * mf
            v1 = v1 * mf
        return a0 + v0, a1 + v1

    def chunk_body(cb, accs):
        a0, a1 = accs
        t0 = cb * LANES
        ichunk = idx_all[b, pl.ds(t0, LANES)]
        olanes = (ichunk & 3) * D
        for j in range(LANES):
            a0, a1 = tok(t0 + j, olanes, j, a0, a1)
        return a0, a1

    a0, a1 = lax.fori_loop(0, nc, chunk_body, (zero, zero))

    t0 = nc * LANES
    t0c = jnp.minimum(t0, L - LANES)
    shift = t0 - t0c
    ichunk = idx_all[b, pl.ds(t0c, LANES)]
    olanes = (ichunk & 3) * D
    remv = jnp.full((LANES,), rem, dtype=jnp.int32)
    for j in range(LANES - 1):
        jv = jnp.full((LANES,), j, dtype=jnp.int32)
        mf = jnp.minimum(jnp.maximum(remv - jv, 0), 1).astype(jnp.float32)
        pos = jnp.minimum(shift + j, LANES - 1)
        row = jnp.minimum(t0 + j, L - 1)
        a0, a1 = tok(row, olanes, pos, a0, a1, mf)

    nvec = jnp.full((LANES,), n, dtype=jnp.int32).astype(jnp.float32)
    out_v[b, pl.ds(0, LANES)] = a0 / nvec
    out_v[b, pl.ds(LANES, LANES)] = a1 / nvec


def _lookup_body(xs_hbm, len_hbm, wlin_hbm, out_hbm,
                 idx_all, qrow_a, qrow_b, rows_a, rows_b, len_v, out_v,
                 sem_a, sem_b):
    c = lax.axis_index("c")
    s = lax.axis_index("s")
    wid = s * 2 + c
    base = wid * BPW

    pltpu.sync_copy(len_hbm.at[pl.ds(base, BPW)], len_v.at[pl.ds(0, BPW)])
    pltpu.sync_copy(xs_hbm.at[pl.ds(base, BPW), :], idx_all)

    _make_qrow(idx_all, qrow_a, 0)
    _gather_batch(wlin_hbm, qrow_a, rows_a, sem_a)
    _make_qrow(idx_all, qrow_b, 1)
    _gather_batch(wlin_hbm, qrow_b, rows_b, sem_b)

    def pair_body(k, carry):
        b0 = 2 * k
        _drain_batch(wlin_hbm, qrow_a, rows_a, sem_a)
        _accum_batch(rows_a, idx_all, len_v, out_v, b0)
        _make_qrow(idx_all, qrow_a, b0 + 2)
        _gather_batch(wlin_hbm, qrow_a, rows_a, sem_a)
        _drain_batch(wlin_hbm, qrow_b, rows_b, sem_b)
        _accum_batch(rows_b, idx_all, len_v, out_v, b0 + 1)
        _make_qrow(idx_all, qrow_b, b0 + 3)
        _gather_batch(wlin_hbm, qrow_b, rows_b, sem_b)
        return carry

    lax.fori_loop(0, BPW // 2 - 1, pair_body, 0)

    _drain_batch(wlin_hbm, qrow_a, rows_a, sem_a)
    _accum_batch(rows_a, idx_all, len_v, out_v, BPW - 2)
    _drain_batch(wlin_hbm, qrow_b, rows_b, sem_b)
    _accum_batch(rows_b, idx_all, len_v, out_v, BPW - 1)

    pltpu.sync_copy(out_v, out_hbm.at[pl.ds(base, BPW)])


@functools.partial(jax.jit, donate_argnums=())
def kernel(xs, xs_len, weight):
    mesh = plsc.VectorSubcoreMesh(core_axis_name="c", subcore_axis_name="s")
    params = pltpu.CompilerParams(use_tc_tiling_on_sc=True,
                                  needs_layout_passes=False)

    relayout = functools.partial(
        pl.kernel,
        mesh=mesh,
        compiler_params=params,
        out_type=jax.ShapeDtypeStruct((VP, DP), jnp.float32),
        scratch_types=[
            pltpu.VMEM((RB, D), jnp.float32),
            pltpu.VMEM((RB, D), jnp.float32),
            pltpu.VMEM((QB, DP), jnp.float32),
            pltpu.VMEM((QB, DP), jnp.float32),
            pltpu.SemaphoreType.DMA,
            pltpu.SemaphoreType.DMA,
            pltpu.SemaphoreType.DMA,
            pltpu.SemaphoreType.DMA,
        ],
    )(_relayout_body)

    lookup = functools.partial(
        pl.kernel,
        mesh=mesh,
        compiler_params=params,
        out_type=jax.ShapeDtypeStruct((B, D), jnp.float32),
        scratch_types=[
            pltpu.VMEM((BPW, L), jnp.int32),
            pltpu.VMEM((L,), jnp.int32),
            pltpu.VMEM((L,), jnp.int32),
            pltpu.VMEM((L, DP), jnp.float32),
            pltpu.VMEM((L, DP), jnp.float32),
            pltpu.VMEM((BPW + LANES,), jnp.int32),
            pltpu.VMEM((BPW, D), jnp.float32),
            pltpu.SemaphoreType.DMA,
            pltpu.SemaphoreType.DMA,
        ],
    )(_lookup_body)

    wlin = relayout(weight.astype(jnp.float32))
    return lookup(xs.astype(jnp.int32), xs_len.astype(jnp.int32), wlin)


# speedup vs baseline: 1.0003x; 1.0003x over previous
import functools

import jax
import jax.numpy as jnp
from jax import lax
from jax.experimental import pallas as pl
from jax.experimental.pallas import tpu as pltpu
from jax.experimental.pallas import tpu_sc as plsc

B, L, V, D = 4096, 200, 1000000, 32
DP = 128
RPP = DP // D
VP = V // RPP
LANES = 16
NUM_WORKERS = 32
BPW = B // NUM_WORKERS
LPAD = 208
C0, C1 = 128, 72

RB = 256
QB = RB // RPP
NPAIR = 61
NBLK_MAIN = 2 * NPAIR * NUM_WORKERS
TAIL0 = NBLK_MAIN * RB
TAIL1 = TAIL0 + RB
TAIL2 = TAIL1 + RB
TAILR = V - TAIL2


def _repack_block(in_v, out_v, nrows):
    def body(rb, carry):
        for j in range(8):
            r = rb * 8 + j
            q = rb * 2 + (j // 4)
            o = (j % 4) * D
            out_v[q, pl.ds(o, LANES)] = in_v[r, pl.ds(0, LANES)]
            out_v[q, pl.ds(o + LANES, LANES)] = in_v[r, pl.ds(LANES, LANES)]
        return carry

    lax.fori_loop(0, nrows // 8, body, 0)


def _relayout_body(w_hbm, wlin_hbm, in_a, in_b, out_a, out_b,
                   sia, sib, soa, sob):
    c = lax.axis_index("c")
    s = lax.axis_index("s")
    wid = s * 2 + c

    def blk_of(i):
        return pl.multiple_of((wid + i * NUM_WORKERS) * RB, RB)

    def fire_in(i, in_v, sem):
        return pltpu.async_copy(w_hbm.at[pl.ds(blk_of(i), RB), :], in_v, sem)

    def wait_in(i, in_v, sem):
        pltpu.make_async_copy(w_hbm.at[pl.ds(blk_of(i), RB), :], in_v, sem).wait()

    def fire_out(i, out_v, sem):
        q0 = pl.multiple_of(blk_of(i) // RPP, QB)
        return pltpu.async_copy(out_v, wlin_hbm.at[pl.ds(q0, QB), :], sem)

    def wait_out(i, out_v, sem):
        q0 = pl.multiple_of(blk_of(i) // RPP, QB)
        pltpu.make_async_copy(
            out_v, wlin_hbm.at[pl.ds(q0, QB), :], sem).wait()

    fire_in(0, in_a, sia)
    fire_in(1, in_b, sib)

    wait_in(0, in_a, sia)
    _repack_block(in_a, out_a, RB)
    fire_in(2, in_a, sia)
    fire_out(0, out_a, soa)
    wait_in(1, in_b, sib)
    _repack_block(in_b, out_b, RB)
    fire_in(3, in_b, sib)
    fire_out(1, out_b, sob)

    def pair(p, carry):
        i0 = 2 * p
        wait_out(i0 - 2, out_a, soa)
        wait_in(i0, in_a, sia)
        _repack_block(in_a, out_a, RB)
        fire_in(i0 + 2, in_a, sia)
        fire_out(i0, out_a, soa)
        wait_out(i0 - 1, out_b, sob)
        wait_in(i0 + 1, in_b, sib)
        _repack_block(in_b, out_b, RB)
        fire_in(i0 + 3, in_b, sib)
        fire_out(i0 + 1, out_b, sob)
        return carry

    lax.fori_loop(1, NPAIR - 1, pair, 0)

    i0 = 2 * (NPAIR - 1)
    wait_out(i0 - 2, out_a, soa)
    wait_in(i0, in_a, sia)
    _repack_block(in_a, out_a, RB)
    fire_out(i0, out_a, soa)
    wait_out(i0 - 1, out_b, sob)
    wait_in(i0 + 1, in_b, sib)
    _repack_block(in_b, out_b, RB)
    fire_out(i0 + 1, out_b, sob)
    wait_out(i0, out_a, soa)
    wait_out(i0 + 1, out_b, sob)

    @pl.when(wid == 0)
    def _():
        pltpu.sync_copy(w_hbm.at[pl.ds(TAIL0, RB), :], in_a)
        _repack_block(in_a, out_a, RB)
        pltpu.sync_copy(out_a, wlin_hbm.at[pl.ds(TAIL0 // RPP, QB), :])

    @pl.when(wid == 1)
    def _():
        pltpu.sync_copy(w_hbm.at[pl.ds(TAIL1, RB), :], in_a)
        _repack_block(in_a, out_a, RB)
        pltpu.sync_copy(out_a, wlin_hbm.at[pl.ds(TAIL1 // RPP, QB), :])

    @pl.when(wid == 2)
    def _():
        pltpu.sync_copy(w_hbm.at[pl.ds(TAIL2, TAILR), :],
                        in_a.at[pl.ds(0, TAILR)])
        _repack_block(in_a, out_a, TAILR)
        pltpu.sync_copy(out_a.at[pl.ds(0, TAILR // RPP)],
                        wlin_hbm.at[pl.ds(TAIL2 // RPP, TAILR // RPP), :])


IOTA = None


def _gather_batch(wlin, qrow, rows, sem):
    cp0 = pltpu.async_copy(
        wlin.at[qrow.at[pl.ds(0, C0)]], rows.at[pl.ds(0, C0)], sem)
    cp1 = pltpu.async_copy(
        wlin.at[qrow.at[pl.ds(C0, C1)]], rows.at[pl.ds(C0, C1)], sem)
    return cp0, cp1


def _drain_batch(wlin, qrow, rows, sem):
    pltpu.make_async_copy(
        wlin.at[qrow.at[pl.ds(0, C0)]], rows.at[pl.ds(0, C0)], sem).wait()
    pltpu.make_async_copy(
        wlin.at[qrow.at[pl.ds(C0, C1)]], rows.at[pl.ds(C0, C1)], sem).wait()


def _make_qrow(idx_all, qrow, b):
    def body(t, carry):
        chunk = idx_all[b, pl.ds(t * LANES, LANES)]
        qrow[pl.ds(t * LANES, LANES)] = lax.shift_right_logical(chunk, 2)
        return carry

    lax.fori_loop(0, L // LANES, body, 0)
    chunk = idx_all[b, pl.ds(L - LANES, LANES)]
    qrow[pl.ds(L - LANES, LANES)] = lax.shift_right_logical(chunk, 2)


def _accum_batch(rows, idx_all, len_v, out_v, b):
    n = len_v[pl.ds(b, LANES)][0]
    nc = n >> 4
    rem = n & 15
    iota = lax.iota(jnp.int32, LANES)
    zero = jnp.zeros((LANES,), jnp.float32)

    def tok(i, olanes, pos, a0, a1, mf=None):
        ol = jnp.take(olanes, jnp.full((LANES,), pos, dtype=jnp.int32))
        lane0 = ol + iota
        row_i = jnp.full((LANES,), i, dtype=jnp.int32)
        v0 = plsc.load_gather(rows, [row_i, lane0])
        v1 = plsc.load_gather(rows, [row_i, lane0 + LANES])
        if mf is not None:
            v0 = v0 * mf
            v1 = v1 * mf
        return a0 + v0, a1 + v1

    def chunk_body(cb, accs):
        a0, a1 = accs
        t0 = cb * LANES
        ichunk = idx_all[b, pl.ds(t0, LANES)]
        olanes = (ichunk & 3) * D
        for j in range(LANES):
            a0, a1 = tok(t0 + j, olanes, j, a0, a1)
        return a0, a1

    a0, a1 = lax.fori_loop(0, nc, chunk_body, (zero, zero))

    t0 = nc * LANES
    t0c = jnp.minimum(t0, L - LANES)
    shift = t0 - t0c
    ichunk = idx_all[b, pl.ds(t0c, LANES)]
    olanes = (ichunk & 3) * D
    remv = jnp.full((LANES,), rem, dtype=jnp.int32)
    for j in range(LANES - 1):
        jv = jnp.full((LANES,), j, dtype=jnp.int32)
        mf = jnp.minimum(jnp.maximum(remv - jv, 0), 1).astype(jnp.float32)
        pos = jnp.minimum(shift + j, LANES - 1)
        row = jnp.minimum(t0 + j, L - 1)
        a0, a1 = tok(row, olanes, pos, a0, a1, mf)

    nvec = jnp.full((LANES,), n, dtype=jnp.int32).astype(jnp.float32)
    out_v[b, pl.ds(0, LANES)] = a0 / nvec
    out_v[b, pl.ds(LANES, LANES)] = a1 / nvec


def _lookup_body(xs_hbm, len_hbm, wlin_hbm, out_hbm,
                 idx_all, qrow_a, qrow_b, rows_a, rows_b, len_v, out_v,
                 sem_a, sem_b):
    c = lax.axis_index("c")
    s = lax.axis_index("s")
    wid = s * 2 + c
    base = wid * BPW

    pltpu.sync_copy(len_hbm.at[pl.ds(base, BPW)], len_v.at[pl.ds(0, BPW)])
    pltpu.sync_copy(xs_hbm.at[pl.ds(base, BPW), :], idx_all)

    _make_qrow(idx_all, qrow_a, 0)
    _gather_batch(wlin_hbm, qrow_a, rows_a, sem_a)
    _make_qrow(idx_all, qrow_b, 1)
    _gather_batch(wlin_hbm, qrow_b, rows_b, sem_b)

    def pair_body(k, carry):
        b0 = 2 * k
        _drain_batch(wlin_hbm, qrow_a, rows_a, sem_a)
        _accum_batch(rows_a, idx_all, len_v, out_v, b0)
        _make_qrow(idx_all, qrow_a, b0 + 2)
        _gather_batch(wlin_hbm, qrow_a, rows_a, sem_a)
        _drain_batch(wlin_hbm, qrow_b, rows_b, sem_b)
        _accum_batch(rows_b, idx_all, len_v, out_v, b0 + 1)
        _make_qrow(idx_all, qrow_b, b0 + 3)
        _gather_batch(wlin_hbm, qrow_b, rows_b, sem_b)
        return carry

    lax.fori_loop(0, BPW // 2 - 1, pair_body, 0)

    _drain_batch(wlin_hbm, qrow_a, rows_a, sem_a)
    _accum_batch(rows_a, idx_all, len_v, out_v, BPW - 2)
    _drain_batch(wlin_hbm, qrow_b, rows_b, sem_b)
    _accum_batch(rows_b, idx_all, len_v, out_v, BPW - 1)

    pltpu.sync_copy(out_v, out_hbm.at[pl.ds(base, BPW)])


@functools.partial(jax.jit, donate_argnums=())
def kernel(xs, xs_len, weight):
    mesh = plsc.VectorSubcoreMesh(core_axis_name="c", subcore_axis_name="s")
    params = pltpu.CompilerParams(use_tc_tiling_on_sc=True,
                                  needs_layout_passes=False)

    relayout = functools.partial(
        pl.kernel,
        mesh=mesh,
        compiler_params=params,
        out_type=jax.ShapeDtypeStruct((VP, DP), jnp.float32),
        scratch_types=[
            pltpu.VMEM((RB, D), jnp.float32),
            pltpu.VMEM((RB, D), jnp.float32),
            pltpu.VMEM((QB, DP), jnp.float32),
            pltpu.VMEM((QB, DP), jnp.float32),
            pltpu.SemaphoreType.DMA,
            pltpu.SemaphoreType.DMA,
            pltpu.SemaphoreType.DMA,
            pltpu.SemaphoreType.DMA,
        ],
    )(_relayout_body)

    lookup = functools.partial(
        pl.kernel,
        mesh=mesh,
        compiler_params=params,
        out_type=jax.ShapeDtypeStruct((B, D), jnp.float32),
        scratch_types=[
            pltpu.VMEM((BPW, L), jnp.int32),
            pltpu.VMEM((L,), jnp.int32),
            pltpu.VMEM((L,), jnp.int32),
            pltpu.VMEM((L, DP), jnp.float32),
            pltpu.VMEM((L, DP), jnp.float32),
            pltpu.VMEM((BPW + LANES,), jnp.int32),
            pltpu.VMEM((BPW, D), jnp.float32),
            pltpu.SemaphoreType.DMA,
            pltpu.SemaphoreType.DMA,
        ],
    )(_lookup_body)

    wview = lax.bitcast_convert_type(
        lax.bitcast_convert_type(weight.astype(jnp.float32), jnp.int32),
        jnp.float32)
    wlin = relayout(wview)
    return lookup(xs.astype(jnp.int32), xs_len.astype(jnp.int32), wlin)


# speedup vs baseline: 1.0719x; 1.0716x over previous
import functools

import jax
import jax.numpy as jnp
from jax import lax
from jax.experimental import pallas as pl
from jax.experimental.pallas import tpu as pltpu
from jax.experimental.pallas import tpu_sc as plsc

B, L, V, D = 4096, 200, 1000000, 32
DP = 128
RPP = DP // D
VP = V // RPP
LANES = 16
NUM_WORKERS = 32
BPW = B // NUM_WORKERS
LPAD = 208
C0, C1 = 128, 72

RB = 256
QB = RB // RPP
NPAIR = 61
NBLK_MAIN = 2 * NPAIR * NUM_WORKERS
TAIL0 = NBLK_MAIN * RB
TAIL1 = TAIL0 + RB
TAIL2 = TAIL1 + RB
TAILR = V - TAIL2


def _repack_block(in_v, out_v, nrows):
    def body(rb, carry):
        for j in range(8):
            r = rb * 8 + j
            q = rb * 2 + (j // 4)
            o = (j % 4) * D
            out_v[q, pl.ds(o, LANES)] = in_v[r, pl.ds(0, LANES)]
            out_v[q, pl.ds(o + LANES, LANES)] = in_v[r, pl.ds(LANES, LANES)]
        return carry

    lax.fori_loop(0, nrows // 8, body, 0)


def _relayout_body(w_hbm, wlin_hbm, in_a, in_b, out_a, out_b,
                   sia, sib, soa, sob):
    c = lax.axis_index("c")
    s = lax.axis_index("s")
    wid = s * 2 + c

    def blk_of(i):
        return pl.multiple_of((wid + i * NUM_WORKERS) * RB, RB)

    def fire_in(i, in_v, sem):
        return pltpu.async_copy(w_hbm.at[pl.ds(blk_of(i), RB), :], in_v, sem)

    def wait_in(i, in_v, sem):
        pltpu.make_async_copy(w_hbm.at[pl.ds(blk_of(i), RB), :], in_v, sem).wait()

    def fire_out(i, out_v, sem):
        q0 = pl.multiple_of(blk_of(i) // RPP, QB)
        return pltpu.async_copy(out_v, wlin_hbm.at[pl.ds(q0, QB), :], sem)

    def wait_out(i, out_v, sem):
        q0 = pl.multiple_of(blk_of(i) // RPP, QB)
        pltpu.make_async_copy(
            out_v, wlin_hbm.at[pl.ds(q0, QB), :], sem).wait()

    fire_in(0, in_a, sia)
    fire_in(1, in_b, sib)

    wait_in(0, in_a, sia)
    _repack_block(in_a, out_a, RB)
    fire_in(2, in_a, sia)
    fire_out(0, out_a, soa)
    wait_in(1, in_b, sib)
    _repack_block(in_b, out_b, RB)
    fire_in(3, in_b, sib)
    fire_out(1, out_b, sob)

    def pair(p, carry):
        i0 = 2 * p
        wait_out(i0 - 2, out_a, soa)
        wait_in(i0, in_a, sia)
        _repack_block(in_a, out_a, RB)
        fire_in(i0 + 2, in_a, sia)
        fire_out(i0, out_a, soa)
        wait_out(i0 - 1, out_b, sob)
        wait_in(i0 + 1, in_b, sib)
        _repack_block(in_b, out_b, RB)
        fire_in(i0 + 3, in_b, sib)
        fire_out(i0 + 1, out_b, sob)
        return carry

    lax.fori_loop(1, NPAIR - 1, pair, 0)

    i0 = 2 * (NPAIR - 1)
    wait_out(i0 - 2, out_a, soa)
    wait_in(i0, in_a, sia)
    _repack_block(in_a, out_a, RB)
    fire_out(i0, out_a, soa)
    wait_out(i0 - 1, out_b, sob)
    wait_in(i0 + 1, in_b, sib)
    _repack_block(in_b, out_b, RB)
    fire_out(i0 + 1, out_b, sob)
    wait_out(i0, out_a, soa)
    wait_out(i0 + 1, out_b, sob)

    @pl.when(wid == 0)
    def _():
        pltpu.sync_copy(w_hbm.at[pl.ds(TAIL0, RB), :], in_a)
        _repack_block(in_a, out_a, RB)
        pltpu.sync_copy(out_a, wlin_hbm.at[pl.ds(TAIL0 // RPP, QB), :])

    @pl.when(wid == 1)
    def _():
        pltpu.sync_copy(w_hbm.at[pl.ds(TAIL1, RB), :], in_a)
        _repack_block(in_a, out_a, RB)
        pltpu.sync_copy(out_a, wlin_hbm.at[pl.ds(TAIL1 // RPP, QB), :])

    @pl.when(wid == 2)
    def _():
        pltpu.sync_copy(w_hbm.at[pl.ds(TAIL2, TAILR), :],
                        in_a.at[pl.ds(0, TAILR)])
        _repack_block(in_a, out_a, TAILR)
        pltpu.sync_copy(out_a.at[pl.ds(0, TAILR // RPP)],
                        wlin_hbm.at[pl.ds(TAIL2 // RPP, TAILR // RPP), :])


IOTA = None


def _gather_batch(wlin, qrow, rows, sem):
    cp0 = pltpu.async_copy(
        wlin.at[qrow.at[pl.ds(0, C0)]], rows.at[pl.ds(0, C0)], sem)
    cp1 = pltpu.async_copy(
        wlin.at[qrow.at[pl.ds(C0, C1)]], rows.at[pl.ds(C0, C1)], sem)
    return cp0, cp1


def _drain_batch(wlin, qrow, rows, sem):
    pltpu.make_async_copy(
        wlin.at[qrow.at[pl.ds(0, C0)]], rows.at[pl.ds(0, C0)], sem).wait()
    pltpu.make_async_copy(
        wlin.at[qrow.at[pl.ds(C0, C1)]], rows.at[pl.ds(C0, C1)], sem).wait()


def _make_qrow(idx_all, qrow, b):
    def body(t, carry):
        chunk = idx_all[b, pl.ds(t * LANES, LANES)]
        qrow[pl.ds(t * LANES, LANES)] = lax.shift_right_logical(chunk, 2)
        return carry

    lax.fori_loop(0, L // LANES, body, 0)
    chunk = idx_all[b, pl.ds(L - LANES, LANES)]
    qrow[pl.ds(L - LANES, LANES)] = lax.shift_right_logical(chunk, 2)


def _accum_batch(rows, idx_all, len_v, out_v, b):
    n = len_v[pl.ds(b, LANES)][0]
    nc = n >> 4
    rem = n & 15
    iota = lax.iota(jnp.int32, LANES)
    zero = jnp.zeros((LANES,), jnp.float32)

    def tok(i, olanes, pos, a0, a1, mf=None):
        ol = jnp.take(olanes, jnp.full((LANES,), pos, dtype=jnp.int32))
        lane0 = ol + iota
        row_i = jnp.full((LANES,), i, dtype=jnp.int32)
        v0 = plsc.load_gather(rows, [row_i, lane0])
        v1 = plsc.load_gather(rows, [row_i, lane0 + LANES])
        if mf is not None:
            v0 = v0 * mf
            v1 = v1 * mf
        return a0 + v0, a1 + v1

    def chunk_body(cb, accs):
        a0, a1 = accs
        t0 = cb * LANES
        ichunk = idx_all[b, pl.ds(t0, LANES)]
        olanes = (ichunk & 3) * D
        for j in range(LANES):
            a0, a1 = tok(t0 + j, olanes, j, a0, a1)
        return a0, a1

    a0, a1 = lax.fori_loop(0, nc, chunk_body, (zero, zero))

    t0 = nc * LANES
    t0c = jnp.minimum(t0, L - LANES)
    shift = t0 - t0c
    ichunk = idx_all[b, pl.ds(t0c, LANES)]
    olanes = (ichunk & 3) * D
    remv = jnp.full((LANES,), rem, dtype=jnp.int32)
    for j in range(LANES - 1):
        jv = jnp.full((LANES,), j, dtype=jnp.int32)
        mf = jnp.minimum(jnp.maximum(remv - jv, 0), 1).astype(jnp.float32)
        pos = jnp.minimum(shift + j, LANES - 1)
        row = jnp.minimum(t0 + j, L - 1)
        a0, a1 = tok(row, olanes, pos, a0, a1, mf)

    nvec = jnp.full((LANES,), n, dtype=jnp.int32).astype(jnp.float32)
    out_v[b, pl.ds(0, LANES)] = a0 / nvec
    out_v[b, pl.ds(LANES, LANES)] = a1 / nvec


def _lookup_body(xs_hbm, len_hbm, wlin_hbm, out_hbm,
                 idx_all, qrow_a, qrow_b, rows_a, rows_b, len_v, out_v,
                 sem_a, sem_b):
    c = lax.axis_index("c")
    s = lax.axis_index("s")
    wid = s * 2 + c
    base = wid * BPW

    pltpu.sync_copy(len_hbm.at[pl.ds(base, BPW)], len_v.at[pl.ds(0, BPW)])
    pltpu.sync_copy(xs_hbm.at[pl.ds(base, BPW), :], idx_all)

    _make_qrow(idx_all, qrow_a, 0)
    _gather_batch(wlin_hbm, qrow_a, rows_a, sem_a)
    _make_qrow(idx_all, qrow_b, 1)
    _gather_batch(wlin_hbm, qrow_b, rows_b, sem_b)

    def pair_body(k, carry):
        b0 = 2 * k
        _drain_batch(wlin_hbm, qrow_a, rows_a, sem_a)
        _accum_batch(rows_a, idx_all, len_v, out_v, b0)
        _make_qrow(idx_all, qrow_a, b0 + 2)
        _gather_batch(wlin_hbm, qrow_a, rows_a, sem_a)
        _drain_batch(wlin_hbm, qrow_b, rows_b, sem_b)
        _accum_batch(rows_b, idx_all, len_v, out_v, b0 + 1)
        _make_qrow(idx_all, qrow_b, b0 + 3)
        _gather_batch(wlin_hbm, qrow_b, rows_b, sem_b)
        return carry

    lax.fori_loop(0, BPW // 2 - 1, pair_body, 0)

    _drain_batch(wlin_hbm, qrow_a, rows_a, sem_a)
    _accum_batch(rows_a, idx_all, len_v, out_v, BPW - 2)
    _drain_batch(wlin_hbm, qrow_b, rows_b, sem_b)
    _accum_batch(rows_b, idx_all, len_v, out_v, BPW - 1)

    pltpu.sync_copy(out_v, out_hbm.at[pl.ds(base, BPW)])


@functools.partial(jax.jit, donate_argnums=())
def kernel(xs, xs_len, weight):
    mesh = plsc.VectorSubcoreMesh(core_axis_name="c", subcore_axis_name="s")
    params = pltpu.CompilerParams(use_tc_tiling_on_sc=True,
                                  needs_layout_passes=False)

    relayout = functools.partial(
        pl.kernel,
        mesh=mesh,
        compiler_params=params,
        out_type=jax.ShapeDtypeStruct((VP, DP), jnp.float32),
        scratch_types=[
            pltpu.VMEM((RB, D), jnp.float32),
            pltpu.VMEM((RB, D), jnp.float32),
            pltpu.VMEM((QB, DP), jnp.float32),
            pltpu.VMEM((QB, DP), jnp.float32),
            pltpu.SemaphoreType.DMA,
            pltpu.SemaphoreType.DMA,
            pltpu.SemaphoreType.DMA,
            pltpu.SemaphoreType.DMA,
        ],
    )(_relayout_body)

    lookup = functools.partial(
        pl.kernel,
        mesh=mesh,
        compiler_params=params,
        out_type=jax.ShapeDtypeStruct((B, D), jnp.float32),
        scratch_types=[
            pltpu.VMEM((BPW, L), jnp.int32),
            pltpu.VMEM((L,), jnp.int32),
            pltpu.VMEM((L,), jnp.int32),
            pltpu.VMEM((L, DP), jnp.float32),
            pltpu.VMEM((L, DP), jnp.float32),
            pltpu.VMEM((BPW + LANES,), jnp.int32),
            pltpu.VMEM((BPW, D), jnp.float32),
            pltpu.SemaphoreType.DMA,
            pltpu.SemaphoreType.DMA,
        ],
    )(_lookup_body)

    wlin = weight.astype(jnp.float32).reshape(VP, DP)
    return lookup(xs.astype(jnp.int32), xs_len.astype(jnp.int32), wlin)


# speedup vs baseline: 1.3069x; 1.2193x over previous
import functools

import jax
import jax.numpy as jnp
from jax import lax
from jax.experimental import pallas as pl
from jax.experimental.pallas import tpu as pltpu
from jax.experimental.pallas import tpu_sc as plsc

B, L, V, D = 4096, 200, 1000000, 32
LANES = 16
NUM_WORKERS = 32
BPW = B // NUM_WORKERS
CH = 40
NCH = L // CH
UNROLL = 4


def _num_chunks(len_v, b):
    n = len_v[pl.ds(b, LANES)][0]
    return n, (n + CH - 1) // CH


def _fire_batch(w_hbm, idx_all, rows, sem, len_v, b):
    _, k = _num_chunks(len_v, b)

    def fire(j, carry):
        off = pl.multiple_of(j * CH, CH)
        pltpu.async_copy(
            w_hbm.at[idx_all.at[b, pl.ds(off, CH)]],
            rows.at[pl.ds(off, CH)], sem)
        return carry

    lax.fori_loop(0, k, fire, 0)


def _drain_batch(w_hbm, idx_all, rows, sem, len_v, b):
    _, k = _num_chunks(len_v, b)

    def drain(j, carry):
        off = pl.multiple_of(j * CH, CH)
        pltpu.make_async_copy(
            w_hbm.at[idx_all.at[b, pl.ds(off, CH)]],
            rows.at[pl.ds(off, CH)], sem).wait()
        return carry

    lax.fori_loop(0, k, drain, 0)


def _accum_batch(rows, len_v, out_v, b):
    n = len_v[pl.ds(b, LANES)][0]
    n4 = n >> 2
    r = n & 3

    def body4(i, accs):
        a0, a1 = accs
        base = i * UNROLL
        for j in range(UNROLL):
            a0 = a0 + rows[base + j, pl.ds(0, LANES)]
            a1 = a1 + rows[base + j, pl.ds(LANES, LANES)]
        return a0, a1

    zero = jnp.zeros((LANES,), jnp.float32)
    a0, a1 = lax.fori_loop(0, n4, body4, (zero, zero))
    tail = n4 * UNROLL
    rvec = jnp.full((LANES,), r, dtype=jnp.int32)
    for j in range(UNROLL - 1):
        jv = jnp.full((LANES,), j, dtype=jnp.int32)
        mf = jnp.minimum(jnp.maximum(rvec - jv, 0), 1).astype(jnp.float32)
        a0 = a0 + rows[tail + j, pl.ds(0, LANES)] * mf
        a1 = a1 + rows[tail + j, pl.ds(LANES, LANES)] * mf
    nvec = jnp.full((LANES,), n, dtype=jnp.int32).astype(jnp.float32)
    out_v[b, pl.ds(0, LANES)] = a0 / nvec
    out_v[b, pl.ds(LANES, LANES)] = a1 / nvec


def _mean_embed_body(xs_hbm, len_hbm, w_hbm, out_hbm,
                     idx_all, rows_a, rows_b, len_v, out_v, sem_a, sem_b):
    c = lax.axis_index("c")
    s = lax.axis_index("s")
    wid = s * 2 + c
    base = wid * BPW

    pltpu.sync_copy(len_hbm.at[pl.ds(base, BPW)], len_v.at[pl.ds(0, BPW)])
    pltpu.sync_copy(xs_hbm.at[pl.ds(base, BPW), :], idx_all)

    _fire_batch(w_hbm, idx_all, rows_a, sem_a, len_v, 0)
    _fire_batch(w_hbm, idx_all, rows_b, sem_b, len_v, 1)

    def pair_body(k, carry):
        b0 = 2 * k
        _drain_batch(w_hbm, idx_all, rows_a, sem_a, len_v, b0)
        _accum_batch(rows_a, len_v, out_v, b0)
        _fire_batch(w_hbm, idx_all, rows_a, sem_a, len_v, b0 + 2)
        _drain_batch(w_hbm, idx_all, rows_b, sem_b, len_v, b0 + 1)
        _accum_batch(rows_b, len_v, out_v, b0 + 1)
        _fire_batch(w_hbm, idx_all, rows_b, sem_b, len_v, b0 + 3)
        return carry

    lax.fori_loop(0, BPW // 2 - 1, pair_body, 0)

    _drain_batch(w_hbm, idx_all, rows_a, sem_a, len_v, BPW - 2)
    _accum_batch(rows_a, len_v, out_v, BPW - 2)
    _drain_batch(w_hbm, idx_all, rows_b, sem_b, len_v, BPW - 1)
    _accum_batch(rows_b, len_v, out_v, BPW - 1)

    pltpu.sync_copy(out_v, out_hbm.at[pl.ds(base, BPW)])


@functools.partial(jax.jit, donate_argnums=())
def kernel(xs, xs_len, weight):
    mesh = plsc.VectorSubcoreMesh(core_axis_name="c", subcore_axis_name="s")
    k = functools.partial(
        pl.kernel,
        mesh=mesh,
        compiler_params=pltpu.CompilerParams(use_tc_tiling_on_sc=False),
        out_type=jax.ShapeDtypeStruct((B, D), jnp.float32),
        scratch_types=[
            pltpu.VMEM((BPW, L), jnp.int32),
            pltpu.VMEM((L, D), jnp.float32),
            pltpu.VMEM((L, D), jnp.float32),
            pltpu.VMEM((BPW + LANES,), jnp.int32),
            pltpu.VMEM((BPW, D), jnp.float32),
            pltpu.SemaphoreType.DMA,
            pltpu.SemaphoreType.DMA,
        ],
    )(_mean_embed_body)
    return k(xs.astype(jnp.int32), xs_len.astype(jnp.int32), weight)
